# Initial kernel scaffold; baseline (speedup 1.0000x reference)
import jax
import jax.numpy as jnp
from jax.experimental import pallas as pl


def kernel(input, rois):
    raise NotImplementedError("write your pallas kernel here")


# speedup vs baseline: 10.3233x; 10.3233x over previous
import functools
import jax
import jax.numpy as jnp
from jax.experimental import pallas as pl
from jax.experimental.pallas import tpu as pltpu

_OH, _OW = 7, 7
_SCALE = 56.0


def _round_half_even_scalar(x):
    t = x + 0.5
    f = t.astype(jnp.int32)
    tie = t == f.astype(jnp.float32)
    odd = (f % 2) == 1
    return f - (tie & odd).astype(jnp.int32)


def _roi_pool_body(rois_ref, x_ref, out_ref, m1_ref):
    H = 56
    W = 56
    b = rois_ref[0, 0, 0].astype(jnp.int32)
    rs_w = _round_half_even_scalar(rois_ref[0, 0, 1] * _SCALE)
    rs_h = _round_half_even_scalar(rois_ref[0, 0, 2] * _SCALE)
    re_w = _round_half_even_scalar(rois_ref[0, 0, 3] * _SCALE)
    re_h = _round_half_even_scalar(rois_ref[0, 0, 4] * _SCALE)
    roi_w = jnp.maximum(re_w - rs_w + 1, 1).astype(jnp.float32)
    roi_h = jnp.maximum(re_h - rs_h + 1, 1).astype(jnp.float32)
    bin_h = roi_h / _OH
    bin_w = roi_w / _OW

    img = x_ref[b]

    io_h = jax.lax.broadcasted_iota(jnp.int32, (H, 1, 1), 0)
    io_w = jax.lax.broadcasted_iota(jnp.int32, (W, 1), 0)

    def floor_nn(z):
        return z.astype(jnp.int32)

    def ceil_nn(z):
        f = z.astype(jnp.int32)
        return f + (z != f.astype(jnp.float32)).astype(jnp.int32)

    hs = []
    he = []
    ws = []
    we = []
    for p in range(_OH):
        s = jnp.clip(floor_nn(jnp.float32(p) * bin_h) + rs_h, 0, H)
        e = jnp.clip(ceil_nn(jnp.float32(p + 1) * bin_h) + rs_h, 0, H)
        hs.append(s)
        he.append(e)
    for p in range(_OW):
        s = jnp.clip(floor_nn(jnp.float32(p) * bin_w) + rs_w, 0, W)
        e = jnp.clip(ceil_nn(jnp.float32(p + 1) * bin_w) + rs_w, 0, W)
        ws.append(s)
        we.append(e)

    neg = jnp.float32(-jnp.inf)

    for ph in range(_OH):
        hm = (io_h >= hs[ph]) & (io_h < he[ph])
        masked = jnp.where(hm, img, neg)
        m1_ref[ph] = jnp.max(masked, axis=0)

    for ph in range(_OH):
        plane = m1_ref[ph]
        h_empty = he[ph] <= hs[ph]
        for pw in range(_OW):
            wm = (io_w >= ws[pw]) & (io_w < we[pw])
            red = jnp.max(jnp.where(wm, plane, neg), axis=0, keepdims=True)
            empty = h_empty | (we[pw] <= ws[pw])
            val = jnp.where(empty, jnp.float32(0.0), red)
            out_ref[0, pl.ds(ph * _OW + pw, 1), :] = val


@jax.jit
def kernel(input, rois):
    N, C, H, W = input.shape
    K = rois.shape[0]
    x4 = jnp.transpose(input, (0, 2, 3, 1))
    rois3 = rois.reshape(K, 1, 5)

    out = pl.pallas_call(
        _roi_pool_body,
        grid=(K,),
        in_specs=[
            pl.BlockSpec((1, 1, 5), lambda i: (i, 0, 0)),
            pl.BlockSpec((N, H, W, C), lambda i: (0, 0, 0, 0)),
        ],
        out_specs=pl.BlockSpec((1, _OH * _OW, C), lambda i: (i, 0, 0)),
        out_shape=jax.ShapeDtypeStruct((K, _OH * _OW, C), jnp.float32),
        scratch_shapes=[pltpu.VMEM((_OH, H, C), jnp.float32)],
    )(rois3, x4)

    return out.reshape(K, _OH, _OW, C).transpose(0, 3, 1, 2)


# speedup vs baseline: 20.9744x; 2.0318x over previous
import jax
import jax.numpy as jnp
from jax import lax
from jax.experimental import pallas as pl
from jax.experimental.pallas import tpu as pltpu
from jax.experimental.pallas import tpu_sc as plsc

_OH, _OW = 7, 7
_SCALE = 56.0
_H = 56
_W = 56
_C = 128
_NC, _NS, _L = 2, 16, 16
_NW = _NC * _NS
_KPAD = 1024
_RPT = _KPAD // _NW
_CH = 4
_NEGINF = float("-inf")


def _round_half_even(x):
    t = x + 0.5
    f = t.astype(jnp.int32)
    tie = t == f.astype(jnp.float32)
    odd = (f & 1) == 1
    return jnp.where(tie & odd, f - 1, f)


def _sc_body(x_hbm, roift_hbm, div7_hbm, out_hbm, rvt, div7, m1, rows_v, outb, sem):
    wid = lax.axis_index("s") * _NC + lax.axis_index("c")
    rbase = wid * _RPT

    pltpu.sync_copy(roift_hbm.at[wid], rvt)
    pltpu.sync_copy(div7_hbm, div7)

    neg = jnp.full((_L,), _NEGINF, dtype=jnp.float32)

    lane = lax.iota(jnp.int32, _L)

    def roi_body(r, carry):
        rrow = rvt[r, pl.ds(0, _L)]
        rounded = _round_half_even(rrow * _SCALE)
        shifted = rounded.at[(lane + 2) & (_L - 1)].get(mode="promise_in_bounds")
        sides = jnp.maximum(shifted - rounded + 1, 1)
        bf = rrow[0]
        bi = bf.astype(jnp.int32)
        b = jnp.where(bi.astype(jnp.float32) > bf, bi - 1, bi)
        rs_w = rounded[1]
        rs_h = rounded[2]
        bw = div7[sides[1], pl.ds(0, _L)][0]
        bh = div7[sides[2], pl.ds(0, _L)][0]
        base = b * (_H * _W)

        def bounds(p, binsz, lo):
            t0 = jnp.float32(p) * binsz
            f0 = t0.astype(jnp.int32)
            fl = jnp.where(f0.astype(jnp.float32) > t0, f0 - 1, f0)
            s = jnp.clip(fl + lo, 0, _H)
            t1 = jnp.float32(p + 1) * binsz
            f1 = t1.astype(jnp.int32)
            ce = jnp.where(f1.astype(jnp.float32) < t1, f1 + 1, f1)
            e = jnp.clip(ce + lo, 0, _H)
            return s, e

        hs, he, ws, we = [], [], [], []
        for p in range(_OH):
            s, e = bounds(p, bh, rs_h)
            hs.append(s)
            he.append(e)
        for p in range(_OW):
            s, e = bounds(p, bw, rs_w)
            ws.append(s)
            we.append(e)
        h0 = hs[0]
        h1 = he[_OH - 1]

        nch = (h1 - h0 + (_CH - 1)) >> 2

        def chunk_body(ci, ccarry):
            c0 = h0 + ci * _CH
            for li in range(_CH):
                line = c0 + li

                @pl.when(line < h1)
                def _():
                    pltpu.make_async_copy(
                        x_hbm.at[pl.ds(pl.multiple_of(base + line * _W, 8), _W)],
                        rows_v.at[pl.ds(li * _W, _W)],
                        sem,
                    ).start()

            for li in range(_CH):
                line = c0 + li

                @pl.when(line < h1)
                def _():
                    pltpu.make_async_copy(
                        x_hbm.at[pl.ds(pl.multiple_of(base + line * _W, 8), _W)],
                        rows_v.at[pl.ds(li * _W, _W)],
                        sem,
                    ).wait()

            for li in range(_CH):
                line = c0 + li

                @pl.when(line < h1)
                def _():
                    for pw in range(_OW):

                        def wstep(w, accs):
                            return tuple(
                                jnp.maximum(
                                    accs[ch],
                                    rows_v[li * _W + w, pl.ds(ch * _L, _L)],
                                )
                                for ch in range(8)
                            )

                        accs = lax.fori_loop(ws[pw], we[pw], wstep, (neg,) * 8)
                        mrow = (line - h0) * _OW + pw
                        for ch in range(8):
                            m1[mrow, pl.ds(ch * _L, _L)] = accs[ch]

            return ccarry

        lax.fori_loop(0, nch, chunk_body, 0)

        for ph in range(_OH):
            for pw in range(_OW):
                k = ph * _OW + pw
                emp = (he[ph] <= hs[ph]) | (we[pw] <= ws[pw])
                init = jnp.full((_L,), jnp.where(emp, jnp.float32(0.0), jnp.float32(_NEGINF)))

                def hstep(h, accs):
                    return tuple(
                        jnp.maximum(
                            accs[ch],
                            m1[(h - h0) * _OW + pw, pl.ds(ch * _L, _L)],
                        )
                        for ch in range(8)
                    )

                accs = lax.fori_loop(hs[ph], he[ph], hstep, (init,) * 8)
                for ch in range(8):
                    outb[k, pl.ds(ch * _L, _L)] = accs[ch]

        pltpu.sync_copy(outb, out_hbm.at[rbase + r])
        return carry

    lax.fori_loop(0, _RPT, roi_body, 0)


@jax.jit
def kernel(input, rois):
    N, C, H, W = input.shape
    K = rois.shape[0]
    x_rows = jnp.transpose(input, (0, 2, 3, 1)).reshape(N * H * W, C)
    roift = jnp.zeros((_KPAD, _L), jnp.float32).at[:K, :5].set(rois)
    roift = roift.reshape(_NW, _RPT, _L)
    div7 = jnp.tile(
        (jnp.arange(64, dtype=jnp.float32) / jnp.float32(_OH))[:, None], (1, _L)
    )

    mesh = plsc.VectorSubcoreMesh(core_axis_name="c", subcore_axis_name="s", num_cores=_NC, num_subcores=_NS)
    out = pl.kernel(
        _sc_body,
        out_type=jax.ShapeDtypeStruct((_KPAD, _OH * _OW, _C), jnp.float32),
        mesh=mesh,
        scratch_types=[
            pltpu.VMEM((_RPT, _L), jnp.float32),
            pltpu.VMEM((64, _L), jnp.float32),
            pltpu.VMEM((_H * _OH, C), jnp.float32),
            pltpu.VMEM((_CH * _W, C), jnp.float32),
            pltpu.VMEM((_OH * _OW, _C), jnp.float32),
            pltpu.SemaphoreType.DMA,
        ],
    )(x_rows, roift, div7)

    return out[:K].transpose(0, 2, 1).reshape(K, C, _OH, _OW)
